# Initial kernel scaffold; baseline (speedup 1.0000x reference)
import jax
import jax.numpy as jnp
from jax.experimental import pallas as pl


def kernel(x, edge_index, edge_weight, H0, H1, H2):
    raise NotImplementedError("write your pallas kernel here")


# speedup vs baseline: 3.0083x; 3.0083x over previous
import dataclasses
import functools

import jax
import jax.numpy as jnp
from jax import lax
from jax.experimental import pallas as pl
from jax.experimental.pallas import tpu as pltpu
from jax.experimental.pallas import tpu_sc as plsc

NC = 2
NS = 16
NW = NC * NS
CHUNK = 128
LANES = 16


def _mm_body(x_ref, h0_ref, h1_ref, h2_ref, y0_ref, y1_ref, y2_ref):
    xb = x_ref[...]
    y0_ref[...] = jnp.dot(xb, h0_ref[...], preferred_element_type=jnp.float32)
    y1_ref[...] = jnp.dot(xb, h1_ref[...], preferred_element_type=jnp.float32)
    y2_ref[...] = jnp.dot(xb, h2_ref[...], preferred_element_type=jnp.float32)


def _projections(x, H0, H1, H2):
    n, d = x.shape
    u = H0.shape[1]
    blk = 1000
    grid = (n // blk,)
    out = jax.ShapeDtypeStruct((n, u), jnp.float32)
    h_spec = pl.BlockSpec((d, u), lambda i: (0, 0))
    row_spec = pl.BlockSpec((blk, d), lambda i: (i, 0))
    return pl.pallas_call(
        _mm_body,
        grid=grid,
        in_specs=[row_spec, h_spec, h_spec, h_spec],
        out_specs=[row_spec, row_spec, row_spec],
        out_shape=[out, out, out],
    )(x, H0, H1, H2)


def _add3_body(y_ref, p0_ref, p1_ref, o_ref):
    o_ref[...] = y_ref[...] + p0_ref[0] + p1_ref[0]


def _combine(y, parts):
    n, d = y.shape
    blk = 1000
    return pl.pallas_call(
        _add3_body,
        grid=(n // blk,),
        in_specs=[
            pl.BlockSpec((blk, d), lambda i: (i, 0)),
            pl.BlockSpec((1, blk, d), lambda i: (0, i, 0)),
            pl.BlockSpec((1, blk, d), lambda i: (1, i, 0)),
        ],
        out_specs=pl.BlockSpec((blk, d), lambda i: (i, 0)),
        out_shape=jax.ShapeDtypeStruct((n, d), jnp.float32),
    )(y, parts, parts)


def _spmm_body(npad, d, k_per_w,
               y_hbm, src_hbm, dst_hbm, w_hbm, out_hbm,
               acc, src_v, dst_v, w_v, rows_v, sem):
    n = npad
    c = lax.axis_index("c")
    s = lax.axis_index("s")
    wid = c * NS + s

    @pl.loop(0, CHUNK)
    def _zero(r):
        for g in range(d // LANES):
            rows_v[r, pl.ds(g * LANES, LANES)] = jnp.zeros((LANES,), jnp.float32)

    rows_per_sub = n // NS
    base = pl.multiple_of(s * rows_per_sub, 8)
    for t in range(rows_per_sub // CHUNK):
        pltpu.sync_copy(rows_v, acc.at[pl.ds(base + t * CHUNK, CHUNK)])
    plsc.subcore_barrier()

    first = pl.multiple_of(wid * k_per_w, 8)
    pltpu.sync_copy(src_hbm.at[pl.ds(first, k_per_w)], src_v)
    pltpu.sync_copy(dst_hbm.at[pl.ds(first, k_per_w)], dst_v)
    pltpu.sync_copy(w_hbm.at[pl.ds(first, k_per_w)], w_v)

    @pl.loop(0, k_per_w)
    def _chunk(j):
        pltpu.async_copy(y_hbm.at[src_v.at[j]], rows_v, sem).wait()

        @pl.loop(0, CHUNK)
        def _scale(r):
            wr = plsc.load_gather(
                w_v, [jnp.full((LANES,), j, jnp.int32),
                      jnp.full((LANES,), r, jnp.int32)])
            for g in range(d // LANES):
                sl = pl.ds(g * LANES, LANES)
                rows_v[r, sl] = rows_v[r, sl] * wr

        pltpu.sync_copy(rows_v, acc.at[dst_v.at[j]], add=True)

    plsc.subcore_barrier()
    pltpu.sync_copy(acc.at[pl.ds(base, rows_per_sub)],
                    out_hbm.at[c, pl.ds(base, rows_per_sub)])


def _spmm(y, src_c, dst_c, w_c):
    n, d = y.shape
    npad = -(-n // (NS * CHUNK)) * (NS * CHUNK)
    k_per_w = src_c.shape[0] // NW
    mesh = plsc.VectorSubcoreMesh(core_axis_name="c", subcore_axis_name="s")
    cp = pltpu.CompilerParams()
    if "needs_layout_passes" in pltpu.CompilerParams.__dataclass_fields__:
        cp = dataclasses.replace(cp, needs_layout_passes=False)
    kern = functools.partial(
        pl.kernel,
        compiler_params=cp,
        out_type=jax.ShapeDtypeStruct((NC, npad, d), jnp.float32),
        mesh=mesh,
        scratch_types=[
            pltpu.VMEM_SHARED((npad, d), jnp.float32),
            pltpu.VMEM((k_per_w, CHUNK), jnp.int32),
            pltpu.VMEM((k_per_w, CHUNK), jnp.int32),
            pltpu.VMEM((k_per_w, CHUNK), jnp.float32),
            pltpu.VMEM((CHUNK, d), jnp.float32),
            pltpu.SemaphoreType.DMA,
        ],
    )(functools.partial(_spmm_body, npad, d, k_per_w))
    return kern(y, src_c, dst_c, w_c)


def kernel(x, edge_index, edge_weight, H0, H1, H2):
    e = edge_weight.shape[0]
    epad = -(-e // (NW * CHUNK * 8)) * (NW * CHUNK * 8)
    pad = epad - e
    dst = jnp.concatenate([edge_index[0], jnp.zeros((pad,), edge_index.dtype)])
    src = jnp.concatenate([edge_index[1], jnp.zeros((pad,), edge_index.dtype)])
    w = jnp.concatenate([edge_weight, jnp.zeros((pad,), edge_weight.dtype)])
    src_c = src.reshape(-1, CHUNK)
    dst_c = dst.reshape(-1, CHUNK)
    w_c = w.reshape(-1, CHUNK)

    y0, y1, y2 = _projections(x, H0, H1, H2)
    p = _spmm(y2, src_c, dst_c, w_c)
    u = _combine(y1, p)
    q = _spmm(u, src_c, dst_c, w_c)
    return _combine(y0, q)


# speedup vs baseline: 11.5576x; 3.8419x over previous
import dataclasses
import functools

import jax
import jax.numpy as jnp
from jax import lax
from jax.experimental import pallas as pl
from jax.experimental.pallas import tpu as pltpu
from jax.experimental.pallas import tpu_sc as plsc

NC = 2
NS = 16
NW = NC * NS
CHUNK = 128
LANES = 16
SBLK = 16


def _mm_body(x_ref, h_ref, y_ref):
    y_ref[...] = jnp.dot(x_ref[...], h_ref[...],
                         preferred_element_type=jnp.float32)


def _project(x, H):
    n, d = x.shape
    u = H.shape[1]
    blk = 1000
    return pl.pallas_call(
        _mm_body,
        grid=(n // blk,),
        in_specs=[pl.BlockSpec((blk, d), lambda i: (i, 0)),
                  pl.BlockSpec((d, u), lambda i: (0, 0))],
        out_specs=pl.BlockSpec((blk, u), lambda i: (i, 0)),
        out_shape=jax.ShapeDtypeStruct((n, u), jnp.float32),
    )(x, H)


def _add3_body(y_ref, p0_ref, p1_ref, o_ref):
    o_ref[...] = y_ref[...] + p0_ref[0] + p1_ref[0]


def _combine(y, parts):
    n, d = y.shape
    blk = 1000
    return pl.pallas_call(
        _add3_body,
        grid=(n // blk,),
        in_specs=[
            pl.BlockSpec((blk, d), lambda i: (i, 0)),
            pl.BlockSpec((1, blk, d), lambda i: (0, i, 0)),
            pl.BlockSpec((1, blk, d), lambda i: (1, i, 0)),
        ],
        out_specs=pl.BlockSpec((blk, d), lambda i: (i, 0)),
        out_shape=jax.ShapeDtypeStruct((n, d), jnp.float32),
    )(y, parts, parts)


def _spmm_body(npad, d, k_per_w,
               y_hbm, src_hbm, dst_hbm, w_hbm, out_hbm,
               acc, src_v, dst_v, w_v, rows0, rows1, gsem0, gsem1):
    c = lax.axis_index("c")
    s = lax.axis_index("s")
    wid = c * NS + s

    @pl.loop(0, CHUNK)
    def _zero(r):
        for g in range(d // LANES):
            rows0[r, pl.ds(g * LANES, LANES)] = jnp.zeros((LANES,), jnp.float32)

    rows_per_sub = npad // NS
    base = pl.multiple_of(s * rows_per_sub, 8)
    for t in range(rows_per_sub // CHUNK):
        pltpu.sync_copy(rows0, acc.at[pl.ds(base + t * CHUNK, CHUNK)])
    plsc.subcore_barrier()

    def scale(rows, j):
        @pl.loop(0, CHUNK, step=2)
        def _scale(r):
            wr0 = plsc.load_gather(
                w_v, [jnp.full((LANES,), j, jnp.int32),
                      jnp.full((LANES,), r, jnp.int32)])
            wr1 = plsc.load_gather(
                w_v, [jnp.full((LANES,), j, jnp.int32),
                      jnp.full((LANES,), r + 1, jnp.int32)])
            for g in range(d // LANES):
                sl = pl.ds(g * LANES, LANES)
                rows[r, sl] = rows[r, sl] * wr0
                rows[r + 1, sl] = rows[r + 1, sl] * wr1

    first = pl.multiple_of(wid * k_per_w, 8)

    @pl.loop(0, k_per_w // SBLK)
    def _blk(b):
        off = pl.multiple_of(first + b * SBLK, 8)
        pltpu.sync_copy(src_hbm.at[pl.ds(off, SBLK)], src_v)
        pltpu.sync_copy(dst_hbm.at[pl.ds(off, SBLK)], dst_v)
        pltpu.sync_copy(w_hbm.at[pl.ds(off, SBLK)], w_v)
        pltpu.async_copy(y_hbm.at[src_v.at[0]], rows0, gsem0)

        @pl.loop(0, SBLK, step=2)
        def _chunk(j):
            pltpu.async_copy(y_hbm.at[src_v.at[j + 1]], rows1, gsem1)
            pltpu.make_async_copy(y_hbm.at[src_v.at[j]], rows0, gsem0).wait()
            scale(rows0, j)
            pltpu.sync_copy(rows0, acc.at[dst_v.at[j]], add=True)

            @pl.when(j + 2 < SBLK)
            def _():
                pltpu.async_copy(y_hbm.at[src_v.at[j + 2]], rows0, gsem0)

            pltpu.make_async_copy(y_hbm.at[src_v.at[j + 1]], rows1, gsem1).wait()
            scale(rows1, j + 1)
            pltpu.sync_copy(rows1, acc.at[dst_v.at[j + 1]], add=True)

    plsc.subcore_barrier()
    pltpu.sync_copy(acc.at[pl.ds(base, rows_per_sub)],
                    out_hbm.at[c, pl.ds(base, rows_per_sub)])


def _spmm(y, src_c, dst_c, w_c):
    n, d = y.shape
    npad = -(-n // (NS * CHUNK)) * (NS * CHUNK)
    k_per_w = src_c.shape[0] // NW
    mesh = plsc.VectorSubcoreMesh(core_axis_name="c", subcore_axis_name="s")
    cp = pltpu.CompilerParams()
    if "needs_layout_passes" in pltpu.CompilerParams.__dataclass_fields__:
        cp = dataclasses.replace(cp, needs_layout_passes=False)
    kern = functools.partial(
        pl.kernel,
        compiler_params=cp,
        out_type=jax.ShapeDtypeStruct((NC, npad, d), jnp.float32),
        mesh=mesh,
        scratch_types=[
            pltpu.VMEM_SHARED((npad, d), jnp.float32),
            pltpu.VMEM((SBLK, CHUNK), jnp.int32),
            pltpu.VMEM((SBLK, CHUNK), jnp.int32),
            pltpu.VMEM((SBLK, CHUNK), jnp.float32),
            pltpu.VMEM((CHUNK, d), jnp.float32),
            pltpu.VMEM((CHUNK, d), jnp.float32),
            pltpu.SemaphoreType.DMA,
            pltpu.SemaphoreType.DMA,
        ],
    )(functools.partial(_spmm_body, npad, d, k_per_w))
    return kern(y, src_c, dst_c, w_c)


def kernel(x, edge_index, edge_weight, H0, H1, H2):
    e = edge_weight.shape[0]
    n = x.shape[0]
    epad = -(-e // (NW * CHUNK * 8)) * (NW * CHUNK * 8)
    pad = epad - e
    spread = jnp.arange(pad, dtype=edge_index.dtype) % n
    dst = jnp.concatenate([edge_index[0], spread])
    src = jnp.concatenate([edge_index[1], spread])
    w = jnp.concatenate([edge_weight, jnp.zeros((pad,), edge_weight.dtype)])
    src_c = src.reshape(-1, CHUNK)
    dst_c = dst.reshape(-1, CHUNK)
    w_c = w.reshape(-1, CHUNK)

    y2 = _project(x, H2)
    p = _spmm(y2, src_c, dst_c, w_c)
    y1 = _project(x, H1)
    u = _combine(y1, p)
    q = _spmm(u, src_c, dst_c, w_c)
    y0 = _project(x, H0)
    return _combine(y0, q)
